# baseline (speedup 1.0000x reference)
import dataclasses

import jax
import jax.numpy as jnp
from jax import lax
from jax.experimental import pallas as pl
from jax.experimental.pallas import tpu as pltpu
from jax.experimental.pallas import tpu_sc as plsc

N = 10000
NPAD = 10240
E = 320000
D = 128

NC = 2
NS = 16
NW = NC * NS

EPT = 9984
EPT_LAST = E - EPT * (NW - 1)
CH = 64
CPT = EPT // CH
CPT_LAST = EPT_LAST // CH
NBUF = 4
ROWS_PER_TILE = NPAD // NS

_mesh = plsc.VectorSubcoreMesh(core_axis_name="c", subcore_axis_name="s")

_sc_params = pltpu.CompilerParams()
if "needs_layout_passes" in pltpu.CompilerParams.__dataclass_fields__:
    _sc_params = dataclasses.replace(_sc_params, needs_layout_passes=False)


def _hist_body(ei_hbm, dego_hbm, degi_hbm, pk_hbm,
               idx_sd, pk_v, ho_v, hi_v, sem):
    cid = lax.axis_index("c")
    sid = lax.axis_index("s")
    wid = sid * NC + cid

    zeros16 = jnp.zeros((16,), jnp.float32)
    ones16 = jnp.ones((16,), jnp.float32)
    sixteen = jnp.full((16,), 16, jnp.int32)

    base = wid * EPT
    pltpu.async_copy(ei_hbm.at[:, pl.ds(base, EPT_LAST)], idx_sd, sem)

    @pl.loop(0, NPAD // 64)
    def _zero(i):
        for q in range(4):
            ho_v[pl.ds(i * 64 + q * 16, 16)] = zeros16
            hi_v[pl.ds(i * 64 + q * 16, 16)] = zeros16

    pltpu.make_async_copy(ei_hbm.at[:, pl.ds(base, EPT_LAST)], idx_sd, sem).wait()

    nvec = jnp.where(wid == NW - 1, EPT_LAST // 16, EPT // 16)

    @pl.loop(0, nvec)
    def _accum(i):
        s = idx_sd[0, pl.ds(i * 16, 16)]
        d = idx_sd[1, pl.ds(i * 16, 16)]
        plsc.addupdate_scatter(ho_v, [s], ones16)
        plsc.addupdate_scatter(hi_v, [d], ones16)
        pk_v[pl.ds(i * 16, 16)] = jnp.bitwise_or(s, lax.shift_left(d, sixteen))

    pltpu.sync_copy(pk_v, pk_hbm.at[wid])
    pltpu.sync_copy(ho_v, dego_hbm.at[wid])
    pltpu.sync_copy(hi_v, degi_hbm.at[wid])


@jax.jit
def _sc_hist(ei):
    k = pl.kernel(
        _hist_body,
        out_type=(
            jax.ShapeDtypeStruct((NW, NPAD), jnp.float32),
            jax.ShapeDtypeStruct((NW, NPAD), jnp.float32),
            jax.ShapeDtypeStruct((NW, EPT_LAST), jnp.int32),
        ),
        mesh=_mesh,
        scratch_types=[
            pltpu.VMEM((2, EPT_LAST), jnp.int32),
            pltpu.VMEM((EPT_LAST,), jnp.int32),
            pltpu.VMEM((NPAD,), jnp.float32),
            pltpu.VMEM((NPAD,), jnp.float32),
            pltpu.SemaphoreType.DMA,
        ],
        compiler_params=_sc_params,
    )
    return k(ei)


def _agg_body(h_hbm, pk_hbm, out_hbm,
              pkb, r0, r1, r2, r3, su0, su1, su2, su3, du0, du1, du2, du3,
              agg_sh, g0, g1, g2, g3, s0, s1, s2, s3):
    cid = lax.axis_index("c")
    sid = lax.axis_index("s")
    wid = sid * NC + cid

    rows = (r0, r1, r2, r3)
    src_u = (su0, su1, su2, su3)
    dst_u = (du0, du1, du2, du3)
    gsem = (g0, g1, g2, g3)
    ssem = (s0, s1, s2, s3)

    zeros16 = jnp.zeros((16,), jnp.float32)
    mask16 = jnp.full((16,), 0xFFFF, jnp.int32)
    sixteen = jnp.full((16,), 16, jnp.int32)

    def unpack(c, p):
        for k8 in range(CH // 16):
            v = pkb[pl.ds(c * CH + k8 * 16, 16)]
            src_u[p][pl.ds(k8 * 16, 16)] = jnp.bitwise_and(v, mask16)
            dst_u[p][pl.ds(k8 * 16, 16)] = lax.shift_right_logical(v, sixteen)

    pltpu.sync_copy(pk_hbm.at[wid], pkb)

    @pl.loop(0, CH)
    def _zero(r):
        for k8 in range(D // 16):
            r0[r, pl.ds(k8 * 16, 16)] = zeros16

    row0 = sid * ROWS_PER_TILE

    @pl.loop(0, ROWS_PER_TILE // CH)
    def _zinit(j):
        pltpu.sync_copy(r0, agg_sh.at[pl.ds(row0 + j * CH, CH)])

    for p in range(NBUF):
        unpack(p, p)
        pltpu.async_copy(h_hbm.at[src_u[p]], rows[p], gsem[p])

    plsc.subcore_barrier()

    def wait_gather(p):
        pltpu.make_async_copy(h_hbm.at[src_u[p]], rows[p], gsem[p]).wait()

    def start_scatter(p):
        pltpu.async_copy(rows[p], agg_sh.at[dst_u[p]], ssem[p], add=True)

    def wait_scatter(p):
        pltpu.make_async_copy(rows[p], agg_sh.at[dst_u[p]], ssem[p]).wait()

    def start_gather(c, p):
        unpack(c, p)
        pltpu.async_copy(h_hbm.at[src_u[p]], rows[p], gsem[p])

    units = jnp.where(wid == NW - 1, CPT_LAST // NBUF, CPT // NBUF)

    @pl.loop(0, units - 1)
    def _unit(j):
        u = j * NBUF
        for p in range(NBUF):
            wait_gather(p)
            start_scatter(p)
        for p in range(NBUF):
            wait_scatter(p)
            start_gather(u + NBUF + p, p)

    for p in range(NBUF):
        wait_gather(p)
        start_scatter(p)
    for p in range(NBUF):
        wait_scatter(p)

    plsc.subcore_barrier()

    pltpu.sync_copy(agg_sh.at[pl.ds(row0, ROWS_PER_TILE)],
                    out_hbm.at[cid, pl.ds(row0, ROWS_PER_TILE)])


@jax.jit
def _sc_agg(h, packed):
    k = pl.kernel(
        _agg_body,
        out_type=jax.ShapeDtypeStruct((NC, NPAD, D), jnp.float32),
        mesh=_mesh,
        scratch_types=[
            pltpu.VMEM((EPT_LAST,), jnp.int32),
            pltpu.VMEM((CH, D), jnp.float32),
            pltpu.VMEM((CH, D), jnp.float32),
            pltpu.VMEM((CH, D), jnp.float32),
            pltpu.VMEM((CH, D), jnp.float32),
            pltpu.VMEM((CH,), jnp.int32),
            pltpu.VMEM((CH,), jnp.int32),
            pltpu.VMEM((CH,), jnp.int32),
            pltpu.VMEM((CH,), jnp.int32),
            pltpu.VMEM((CH,), jnp.int32),
            pltpu.VMEM((CH,), jnp.int32),
            pltpu.VMEM((CH,), jnp.int32),
            pltpu.VMEM((CH,), jnp.int32),
            pltpu.VMEM_SHARED((NPAD, D), jnp.float32),
            pltpu.SemaphoreType.DMA,
            pltpu.SemaphoreType.DMA,
            pltpu.SemaphoreType.DMA,
            pltpu.SemaphoreType.DMA,
            pltpu.SemaphoreType.DMA,
            pltpu.SemaphoreType.DMA,
            pltpu.SemaphoreType.DMA,
            pltpu.SemaphoreType.DMA,
        ],
        compiler_params=_sc_params,
    )
    return k(h, packed)


BLK = 1024


def _mm_body(x_ref, w_ref, degp_ref, o_ref):
    deg = jnp.sum(degp_ref[...], axis=0)
    norm = lax.rsqrt(jnp.maximum(deg, 1.0))
    z = jnp.dot(x_ref[...], w_ref[...],
                preferred_element_type=jnp.float32,
                precision=lax.Precision.HIGHEST)
    o_ref[...] = z * norm[:, None]


@jax.jit
def _tc_matmul_scale(x, w, degp):
    return pl.pallas_call(
        _mm_body,
        out_shape=jax.ShapeDtypeStruct((NPAD, D), jnp.float32),
        grid=(NPAD // BLK,),
        in_specs=[
            pl.BlockSpec((BLK, D), lambda i: (i, 0)),
            pl.BlockSpec((D, D), lambda i: (0, 0)),
            pl.BlockSpec((NW, BLK), lambda i: (0, i)),
        ],
        out_specs=pl.BlockSpec((BLK, D), lambda i: (i, 0)),
    )(x, w, degp)


def _post_body(p_ref, degp_ref, b_ref, o_ref):
    deg = jnp.sum(degp_ref[...], axis=0)
    norm = lax.rsqrt(jnp.maximum(deg, 1.0))
    agg = p_ref[0] + p_ref[1]
    o_ref[...] = agg * norm[:, None] + b_ref[...][None, :]


@jax.jit
def _tc_post(p, degp, b):
    return pl.pallas_call(
        _post_body,
        out_shape=jax.ShapeDtypeStruct((N, D), jnp.float32),
        grid=(NPAD // BLK,),
        in_specs=[
            pl.BlockSpec((NC, BLK, D), lambda i: (0, i, 0)),
            pl.BlockSpec((NW, BLK), lambda i: (0, i)),
            pl.BlockSpec((D,), lambda i: (0,)),
        ],
        out_specs=pl.BlockSpec((BLK, D), lambda i: (i, 0)),
    )(p, degp, b)


def kernel(ndata, edge_index, W, b):
    ei = edge_index.astype(jnp.int32)
    dego_p, degi_p, packed = _sc_hist(ei)
    h = _tc_matmul_scale(ndata, W, dego_p)
    p = _sc_agg(h, packed)
    return _tc_post(p, degi_p, b)


# speedup vs baseline: 1.0434x; 1.0434x over previous
import dataclasses

import jax
import jax.numpy as jnp
from jax import lax
from jax.experimental import pallas as pl
from jax.experimental.pallas import tpu as pltpu
from jax.experimental.pallas import tpu_sc as plsc

N = 10000
NPAD = 10240
E = 320000
D = 128

NC = 2
NS = 16
NW = NC * NS

EPT = 9984
EPT_LAST = E - EPT * (NW - 1)
CH = 64
CPT = EPT // CH
CPT_LAST = EPT_LAST // CH
NBUF = 4
ROWS_PER_TILE = NPAD // NS

_mesh = plsc.VectorSubcoreMesh(core_axis_name="c", subcore_axis_name="s")

_sc_params = pltpu.CompilerParams()
if "needs_layout_passes" in pltpu.CompilerParams.__dataclass_fields__:
    _sc_params = dataclasses.replace(_sc_params, needs_layout_passes=False)


def _hist_body(ei_hbm, dego_hbm, degi_hbm, pk_hbm,
               idx_sd, pk_v, ho_v, hi_v, sem):
    cid = lax.axis_index("c")
    sid = lax.axis_index("s")
    wid = sid * NC + cid

    zeros16 = jnp.zeros((16,), jnp.float32)
    ones16 = jnp.ones((16,), jnp.float32)
    sixteen = jnp.full((16,), 16, jnp.int32)

    base = wid * EPT
    pltpu.async_copy(ei_hbm.at[:, pl.ds(base, EPT_LAST)], idx_sd, sem)

    @pl.loop(0, NPAD // 64)
    def _zero(i):
        for q in range(4):
            ho_v[pl.ds(i * 64 + q * 16, 16)] = zeros16
            hi_v[pl.ds(i * 64 + q * 16, 16)] = zeros16

    pltpu.make_async_copy(ei_hbm.at[:, pl.ds(base, EPT_LAST)], idx_sd, sem).wait()

    nvec = jnp.where(wid == NW - 1, EPT_LAST // 16, EPT // 16)

    @pl.loop(0, nvec)
    def _accum(i):
        s = idx_sd[0, pl.ds(i * 16, 16)]
        d = idx_sd[1, pl.ds(i * 16, 16)]
        plsc.addupdate_scatter(ho_v, [s], ones16)
        plsc.addupdate_scatter(hi_v, [d], ones16)
        pk_v[pl.ds(i * 16, 16)] = jnp.bitwise_or(s, lax.shift_left(d, sixteen))

    pltpu.sync_copy(pk_v, pk_hbm.at[wid])
    pltpu.sync_copy(ho_v, dego_hbm.at[wid])
    pltpu.sync_copy(hi_v, degi_hbm.at[wid])


@jax.jit
def _sc_hist(ei):
    k = pl.kernel(
        _hist_body,
        out_type=(
            jax.ShapeDtypeStruct((NW, NPAD), jnp.float32),
            jax.ShapeDtypeStruct((NW, NPAD), jnp.float32),
            jax.ShapeDtypeStruct((NW, EPT_LAST), jnp.int32),
        ),
        mesh=_mesh,
        scratch_types=[
            pltpu.VMEM((2, EPT_LAST), jnp.int32),
            pltpu.VMEM((EPT_LAST,), jnp.int32),
            pltpu.VMEM((NPAD,), jnp.float32),
            pltpu.VMEM((NPAD,), jnp.float32),
            pltpu.SemaphoreType.DMA,
        ],
        compiler_params=_sc_params,
    )
    return k(ei)


def _agg_body(h_hbm, pk_hbm, out_hbm,
              pkb, r0, r1, r2, r3, su0, su1, su2, su3, du0, du1, du2, du3,
              agg_sh, g0, g1, g2, g3, s0, s1, s2, s3):
    cid = lax.axis_index("c")
    sid = lax.axis_index("s")
    wid = sid * NC + cid

    rows = (r0, r1, r2, r3)
    src_u = (su0, su1, su2, su3)
    dst_u = (du0, du1, du2, du3)
    gsem = (g0, g1, g2, g3)
    ssem = (s0, s1, s2, s3)

    zeros16 = jnp.zeros((16,), jnp.float32)
    mask16 = jnp.full((16,), 0xFFFF, jnp.int32)
    sixteen = jnp.full((16,), 16, jnp.int32)

    def unpack(c, p):
        for k8 in range(CH // 16):
            v = pkb[pl.ds(c * CH + k8 * 16, 16)]
            src_u[p][pl.ds(k8 * 16, 16)] = jnp.bitwise_and(v, mask16)
            dst_u[p][pl.ds(k8 * 16, 16)] = lax.shift_right_logical(v, sixteen)

    pltpu.sync_copy(pk_hbm.at[wid], pkb)

    @pl.loop(0, CH)
    def _zero(r):
        for k8 in range(D // 16):
            r0[r, pl.ds(k8 * 16, 16)] = zeros16

    row0 = sid * ROWS_PER_TILE

    @pl.loop(0, ROWS_PER_TILE // CH)
    def _zinit(j):
        pltpu.sync_copy(r0, agg_sh.at[pl.ds(row0 + j * CH, CH)])

    for p in range(NBUF):
        unpack(p, p)
        pltpu.async_copy(h_hbm.at[src_u[p]], rows[p], gsem[p])

    plsc.subcore_barrier()

    def wait_gather(p):
        pltpu.make_async_copy(h_hbm.at[src_u[p]], rows[p], gsem[p]).wait()

    def start_scatter(p):
        pltpu.async_copy(rows[p], agg_sh.at[dst_u[p]], ssem[p], add=True)

    def wait_scatter(p):
        pltpu.make_async_copy(rows[p], agg_sh.at[dst_u[p]], ssem[p]).wait()

    def start_gather(c, p):
        unpack(c, p)
        pltpu.async_copy(h_hbm.at[src_u[p]], rows[p], gsem[p])

    units = jnp.where(wid == NW - 1, CPT_LAST // NBUF, CPT // NBUF)

    @pl.loop(0, units - 1)
    def _unit(j):
        u = j * NBUF
        for p in range(NBUF):
            wait_gather(p)
            start_scatter(p)
        for p in range(NBUF):
            wait_scatter(p)
            start_gather(u + NBUF + p, p)

    for p in range(NBUF):
        wait_gather(p)
        start_scatter(p)
    for p in range(NBUF):
        wait_scatter(p)

    plsc.subcore_barrier()

    pltpu.sync_copy(agg_sh.at[pl.ds(row0, ROWS_PER_TILE)],
                    out_hbm.at[cid, pl.ds(row0, ROWS_PER_TILE)])


@jax.jit
def _sc_agg(h, packed):
    k = pl.kernel(
        _agg_body,
        out_type=jax.ShapeDtypeStruct((NC, NPAD, D), jnp.float32),
        mesh=_mesh,
        scratch_types=[
            pltpu.VMEM((EPT_LAST,), jnp.int32),
            pltpu.VMEM((CH, D), jnp.float32),
            pltpu.VMEM((CH, D), jnp.float32),
            pltpu.VMEM((CH, D), jnp.float32),
            pltpu.VMEM((CH, D), jnp.float32),
            pltpu.VMEM((CH,), jnp.int32),
            pltpu.VMEM((CH,), jnp.int32),
            pltpu.VMEM((CH,), jnp.int32),
            pltpu.VMEM((CH,), jnp.int32),
            pltpu.VMEM((CH,), jnp.int32),
            pltpu.VMEM((CH,), jnp.int32),
            pltpu.VMEM((CH,), jnp.int32),
            pltpu.VMEM((CH,), jnp.int32),
            pltpu.VMEM_SHARED((NPAD, D), jnp.float32),
            pltpu.SemaphoreType.DMA,
            pltpu.SemaphoreType.DMA,
            pltpu.SemaphoreType.DMA,
            pltpu.SemaphoreType.DMA,
            pltpu.SemaphoreType.DMA,
            pltpu.SemaphoreType.DMA,
            pltpu.SemaphoreType.DMA,
            pltpu.SemaphoreType.DMA,
        ],
        compiler_params=_sc_params,
    )
    return k(h, packed)


BLK = 2560


def _mm_body(x_ref, w_ref, degp_ref, o_ref):
    deg = jnp.sum(degp_ref[...], axis=0)
    norm = lax.rsqrt(jnp.maximum(deg, 1.0))
    z = jnp.dot(x_ref[...], w_ref[...],
                preferred_element_type=jnp.float32,
                precision=lax.Precision.HIGHEST)
    o_ref[...] = z * norm[:, None]


@jax.jit
def _tc_matmul_scale(x, w, degp):
    return pl.pallas_call(
        _mm_body,
        out_shape=jax.ShapeDtypeStruct((NPAD, D), jnp.float32),
        grid=(NPAD // BLK,),
        in_specs=[
            pl.BlockSpec((BLK, D), lambda i: (i, 0)),
            pl.BlockSpec((D, D), lambda i: (0, 0)),
            pl.BlockSpec((NW, BLK), lambda i: (0, i)),
        ],
        out_specs=pl.BlockSpec((BLK, D), lambda i: (i, 0)),
    )(x, w, degp)


def _post_body(p_ref, degp_ref, b_ref, o_ref):
    deg = jnp.sum(degp_ref[...], axis=0)
    norm = lax.rsqrt(jnp.maximum(deg, 1.0))
    agg = p_ref[0] + p_ref[1]
    o_ref[...] = agg * norm[:, None] + b_ref[...][None, :]


@jax.jit
def _tc_post(p, degp, b):
    return pl.pallas_call(
        _post_body,
        out_shape=jax.ShapeDtypeStruct((N, D), jnp.float32),
        grid=(NPAD // BLK,),
        in_specs=[
            pl.BlockSpec((NC, BLK, D), lambda i: (0, i, 0)),
            pl.BlockSpec((NW, BLK), lambda i: (0, i)),
            pl.BlockSpec((D,), lambda i: (0,)),
        ],
        out_specs=pl.BlockSpec((BLK, D), lambda i: (i, 0)),
    )(p, degp, b)


def kernel(ndata, edge_index, W, b):
    ei = edge_index.astype(jnp.int32)
    dego_p, degi_p, packed = _sc_hist(ei)
    h = _tc_matmul_scale(ndata, W, dego_p)
    p = _sc_agg(h, packed)
    return _tc_post(p, degi_p, b)


# speedup vs baseline: 1.0550x; 1.0111x over previous
import dataclasses

import jax
import jax.numpy as jnp
from jax import lax
from jax.experimental import pallas as pl
from jax.experimental.pallas import tpu as pltpu
from jax.experimental.pallas import tpu_sc as plsc

N = 10000
NPAD = 10240
E = 320000
D = 128

NC = 2
NS = 16
NW = NC * NS

EPT = 9984
EPT_LAST = E - EPT * (NW - 1)
CH = 64
CPT = EPT // CH
CPT_LAST = EPT_LAST // CH
NBUF = 4
ROWS_PER_TILE = NPAD // NS

_mesh = plsc.VectorSubcoreMesh(core_axis_name="c", subcore_axis_name="s")

_sc_params = pltpu.CompilerParams()
if "needs_layout_passes" in pltpu.CompilerParams.__dataclass_fields__:
    _sc_params = dataclasses.replace(_sc_params, needs_layout_passes=False)


def _hist_body(ei_hbm, dego_hbm, degi_hbm, pk_hbm,
               idx_sd, pk_v, ho_v, hi_v, sem):
    cid = lax.axis_index("c")
    sid = lax.axis_index("s")
    wid = sid * NC + cid

    zeros16 = jnp.zeros((16,), jnp.float32)
    ones16 = jnp.ones((16,), jnp.float32)
    sixteen = jnp.full((16,), 16, jnp.int32)

    base = wid * EPT
    pltpu.async_copy(ei_hbm.at[:, pl.ds(base, EPT_LAST)], idx_sd, sem)

    @pl.loop(0, NPAD // 64)
    def _zero(i):
        for q in range(4):
            ho_v[pl.ds(i * 64 + q * 16, 16)] = zeros16
            hi_v[pl.ds(i * 64 + q * 16, 16)] = zeros16

    pltpu.make_async_copy(ei_hbm.at[:, pl.ds(base, EPT_LAST)], idx_sd, sem).wait()

    nvec = jnp.where(wid == NW - 1, EPT_LAST // 32, EPT // 32)

    @pl.loop(0, nvec)
    def _accum(i):
        for q in range(2):
            s = idx_sd[0, pl.ds(i * 32 + q * 16, 16)]
            d = idx_sd[1, pl.ds(i * 32 + q * 16, 16)]
            plsc.addupdate_scatter(ho_v, [s], ones16)
            plsc.addupdate_scatter(hi_v, [d], ones16)
            pk_v[pl.ds(i * 32 + q * 16, 16)] = jnp.bitwise_or(
                s, lax.shift_left(d, sixteen))

    pltpu.sync_copy(pk_v, pk_hbm.at[wid])
    pltpu.sync_copy(ho_v, dego_hbm.at[wid])
    pltpu.sync_copy(hi_v, degi_hbm.at[wid])


@jax.jit
def _sc_hist(ei):
    k = pl.kernel(
        _hist_body,
        out_type=(
            jax.ShapeDtypeStruct((NW, NPAD), jnp.float32),
            jax.ShapeDtypeStruct((NW, NPAD), jnp.float32),
            jax.ShapeDtypeStruct((NW, EPT_LAST), jnp.int32),
        ),
        mesh=_mesh,
        scratch_types=[
            pltpu.VMEM((2, EPT_LAST), jnp.int32),
            pltpu.VMEM((EPT_LAST,), jnp.int32),
            pltpu.VMEM((NPAD,), jnp.float32),
            pltpu.VMEM((NPAD,), jnp.float32),
            pltpu.SemaphoreType.DMA,
        ],
        compiler_params=_sc_params,
    )
    return k(ei)


def _agg_body(h_hbm, pk_hbm, out_hbm,
              pkb, r0, r1, r2, r3, su0, su1, su2, su3, du0, du1, du2, du3,
              agg_sh, g0, g1, g2, g3, s0, s1, s2, s3):
    cid = lax.axis_index("c")
    sid = lax.axis_index("s")
    wid = sid * NC + cid

    rows = (r0, r1, r2, r3)
    src_u = (su0, su1, su2, su3)
    dst_u = (du0, du1, du2, du3)
    gsem = (g0, g1, g2, g3)
    ssem = (s0, s1, s2, s3)

    zeros16 = jnp.zeros((16,), jnp.float32)
    mask16 = jnp.full((16,), 0xFFFF, jnp.int32)
    sixteen = jnp.full((16,), 16, jnp.int32)

    def unpack(c, p):
        for k8 in range(CH // 16):
            v = pkb[pl.ds(c * CH + k8 * 16, 16)]
            src_u[p][pl.ds(k8 * 16, 16)] = jnp.bitwise_and(v, mask16)
            dst_u[p][pl.ds(k8 * 16, 16)] = lax.shift_right_logical(v, sixteen)

    pltpu.sync_copy(pk_hbm.at[wid], pkb)

    @pl.loop(0, CH)
    def _zero(r):
        for k8 in range(D // 16):
            r0[r, pl.ds(k8 * 16, 16)] = zeros16

    row0 = sid * ROWS_PER_TILE

    @pl.loop(0, ROWS_PER_TILE // CH)
    def _zinit(j):
        pltpu.sync_copy(r0, agg_sh.at[pl.ds(row0 + j * CH, CH)])

    for p in range(NBUF):
        unpack(p, p)
        pltpu.async_copy(h_hbm.at[src_u[p]], rows[p], gsem[p])

    plsc.subcore_barrier()

    def wait_gather(p):
        pltpu.make_async_copy(h_hbm.at[src_u[p]], rows[p], gsem[p]).wait()

    def start_scatter(p):
        pltpu.async_copy(rows[p], agg_sh.at[dst_u[p]], ssem[p], add=True)

    def wait_scatter(p):
        pltpu.make_async_copy(rows[p], agg_sh.at[dst_u[p]], ssem[p]).wait()

    def start_gather(c, p):
        unpack(c, p)
        pltpu.async_copy(h_hbm.at[src_u[p]], rows[p], gsem[p])

    units = jnp.where(wid == NW - 1, CPT_LAST // NBUF, CPT // NBUF)

    @pl.loop(0, units - 1)
    def _unit(j):
        u = j * NBUF
        for p in range(NBUF):
            wait_gather(p)
            start_scatter(p)
        for p in range(NBUF):
            wait_scatter(p)
            start_gather(u + NBUF + p, p)

    for p in range(NBUF):
        wait_gather(p)
        start_scatter(p)
    for p in range(NBUF):
        wait_scatter(p)

    plsc.subcore_barrier()

    pltpu.sync_copy(agg_sh.at[pl.ds(row0, ROWS_PER_TILE)],
                    out_hbm.at[cid, pl.ds(row0, ROWS_PER_TILE)])


@jax.jit
def _sc_agg(h, packed):
    k = pl.kernel(
        _agg_body,
        out_type=jax.ShapeDtypeStruct((NC, NPAD, D), jnp.float32),
        mesh=_mesh,
        scratch_types=[
            pltpu.VMEM((EPT_LAST,), jnp.int32),
            pltpu.VMEM((CH, D), jnp.float32),
            pltpu.VMEM((CH, D), jnp.float32),
            pltpu.VMEM((CH, D), jnp.float32),
            pltpu.VMEM((CH, D), jnp.float32),
            pltpu.VMEM((CH,), jnp.int32),
            pltpu.VMEM((CH,), jnp.int32),
            pltpu.VMEM((CH,), jnp.int32),
            pltpu.VMEM((CH,), jnp.int32),
            pltpu.VMEM((CH,), jnp.int32),
            pltpu.VMEM((CH,), jnp.int32),
            pltpu.VMEM((CH,), jnp.int32),
            pltpu.VMEM((CH,), jnp.int32),
            pltpu.VMEM_SHARED((NPAD, D), jnp.float32),
            pltpu.SemaphoreType.DMA,
            pltpu.SemaphoreType.DMA,
            pltpu.SemaphoreType.DMA,
            pltpu.SemaphoreType.DMA,
            pltpu.SemaphoreType.DMA,
            pltpu.SemaphoreType.DMA,
            pltpu.SemaphoreType.DMA,
            pltpu.SemaphoreType.DMA,
        ],
        compiler_params=_sc_params,
    )
    return k(h, packed)


BLK = 2560


def _mm_body(x_ref, w_ref, degp_ref, o_ref):
    deg = jnp.sum(degp_ref[...], axis=0)
    norm = lax.rsqrt(jnp.maximum(deg, 1.0))
    z = jnp.dot(x_ref[...], w_ref[...],
                preferred_element_type=jnp.float32)
    o_ref[...] = z * norm[:, None]


@jax.jit
def _tc_matmul_scale(x, w, degp):
    return pl.pallas_call(
        _mm_body,
        out_shape=jax.ShapeDtypeStruct((NPAD, D), jnp.float32),
        grid=(NPAD // BLK,),
        in_specs=[
            pl.BlockSpec((BLK, D), lambda i: (i, 0)),
            pl.BlockSpec((D, D), lambda i: (0, 0)),
            pl.BlockSpec((NW, BLK), lambda i: (0, i)),
        ],
        out_specs=pl.BlockSpec((BLK, D), lambda i: (i, 0)),
    )(x, w, degp)


def _post_body(p_ref, degp_ref, b_ref, o_ref):
    deg = jnp.sum(degp_ref[...], axis=0)
    norm = lax.rsqrt(jnp.maximum(deg, 1.0))
    agg = p_ref[0] + p_ref[1]
    o_ref[...] = agg * norm[:, None] + b_ref[...][None, :]


@jax.jit
def _tc_post(p, degp, b):
    return pl.pallas_call(
        _post_body,
        out_shape=jax.ShapeDtypeStruct((N, D), jnp.float32),
        grid=(NPAD // BLK,),
        in_specs=[
            pl.BlockSpec((NC, BLK, D), lambda i: (0, i, 0)),
            pl.BlockSpec((NW, BLK), lambda i: (0, i)),
            pl.BlockSpec((D,), lambda i: (0,)),
        ],
        out_specs=pl.BlockSpec((BLK, D), lambda i: (i, 0)),
    )(p, degp, b)


def kernel(ndata, edge_index, W, b):
    ei = edge_index.astype(jnp.int32)
    dego_p, degi_p, packed = _sc_hist(ei)
    h = _tc_matmul_scale(ndata, W, dego_p)
    p = _sc_agg(h, packed)
    return _tc_post(p, degi_p, b)


# speedup vs baseline: 1.0644x; 1.0089x over previous
import dataclasses

import jax
import jax.numpy as jnp
from jax import lax
from jax.experimental import pallas as pl
from jax.experimental.pallas import tpu as pltpu
from jax.experimental.pallas import tpu_sc as plsc

N = 10000
NPAD = 10240
E = 320000
D = 128

NC = 2
NS = 16
NW = NC * NS

EPT = 9984
EPT_LAST = E - EPT * (NW - 1)
CH = 64
CPT = EPT // CH
CPT_LAST = EPT_LAST // CH
NBUF = 4
ROWS_PER_TILE = NPAD // NS

_mesh = plsc.VectorSubcoreMesh(core_axis_name="c", subcore_axis_name="s")

_sc_params = pltpu.CompilerParams()
if "needs_layout_passes" in pltpu.CompilerParams.__dataclass_fields__:
    _sc_params = dataclasses.replace(_sc_params, needs_layout_passes=False)


def _hist_body(ei_hbm, dego_hbm, degi_hbm, pk_hbm,
               idx_sd, pk_v, ho_v, hi_v, sem):
    cid = lax.axis_index("c")
    sid = lax.axis_index("s")
    wid = sid * NC + cid

    zeros16 = jnp.zeros((16,), jnp.float32)
    ones16 = jnp.ones((16,), jnp.float32)
    sixteen = jnp.full((16,), 16, jnp.int32)

    base = wid * EPT
    pltpu.async_copy(ei_hbm.at[:, pl.ds(base, EPT_LAST)], idx_sd, sem)

    @pl.loop(0, NPAD // 64)
    def _zero(i):
        for q in range(4):
            ho_v[pl.ds(i * 64 + q * 16, 16)] = zeros16
            hi_v[pl.ds(i * 64 + q * 16, 16)] = zeros16

    pltpu.make_async_copy(ei_hbm.at[:, pl.ds(base, EPT_LAST)], idx_sd, sem).wait()

    nvec = jnp.where(wid == NW - 1, EPT_LAST // 32, EPT // 32)

    @pl.loop(0, nvec)
    def _accum(i):
        for q in range(2):
            s = idx_sd[0, pl.ds(i * 32 + q * 16, 16)]
            d = idx_sd[1, pl.ds(i * 32 + q * 16, 16)]
            plsc.addupdate_scatter(ho_v, [s], ones16)
            plsc.addupdate_scatter(hi_v, [d], ones16)
            pk_v[pl.ds(i * 32 + q * 16, 16)] = jnp.bitwise_or(
                s, lax.shift_left(d, sixteen))

    pltpu.sync_copy(pk_v, pk_hbm.at[wid])
    pltpu.sync_copy(ho_v, dego_hbm.at[wid])
    pltpu.sync_copy(hi_v, degi_hbm.at[wid])


@jax.jit
def _sc_hist(ei):
    k = pl.kernel(
        _hist_body,
        out_type=(
            jax.ShapeDtypeStruct((NW, NPAD), jnp.float32),
            jax.ShapeDtypeStruct((NW, NPAD), jnp.float32),
            jax.ShapeDtypeStruct((NW, EPT_LAST), jnp.int32),
        ),
        mesh=_mesh,
        scratch_types=[
            pltpu.VMEM((2, EPT_LAST), jnp.int32),
            pltpu.VMEM((EPT_LAST,), jnp.int32),
            pltpu.VMEM((NPAD,), jnp.float32),
            pltpu.VMEM((NPAD,), jnp.float32),
            pltpu.SemaphoreType.DMA,
        ],
        compiler_params=_sc_params,
    )
    return k(ei)


def _agg_body(h_hbm, pk_hbm, out_hbm,
              pkb, r0, r1, r2, r3, su0, su1, su2, su3, du0, du1, du2, du3,
              agg_sh, g0, g1, g2, g3, s0, s1, s2, s3):
    cid = lax.axis_index("c")
    sid = lax.axis_index("s")
    wid = sid * NC + cid

    rows = (r0, r1, r2, r3)
    src_u = (su0, su1, su2, su3)
    dst_u = (du0, du1, du2, du3)
    gsem = (g0, g1, g2, g3)
    ssem = (s0, s1, s2, s3)

    zeros16 = jnp.zeros((16,), jnp.float32)
    mask16 = jnp.full((16,), 0xFFFF, jnp.int32)
    sixteen = jnp.full((16,), 16, jnp.int32)

    def unpack(c, p):
        for k8 in range(CH // 16):
            v = pkb[pl.ds(c * CH + k8 * 16, 16)]
            src_u[p][pl.ds(k8 * 16, 16)] = jnp.bitwise_and(v, mask16)
            dst_u[p][pl.ds(k8 * 16, 16)] = lax.shift_right_logical(v, sixteen)

    pltpu.sync_copy(pk_hbm.at[wid], pkb)

    for p in range(NBUF):
        unpack(p, p)
    for p in range(1, NBUF):
        pltpu.async_copy(h_hbm.at[src_u[p]], rows[p], gsem[p])

    @pl.loop(0, CH)
    def _zero(r):
        for k8 in range(D // 16):
            r0[r, pl.ds(k8 * 16, 16)] = zeros16

    row0 = sid * ROWS_PER_TILE
    for j in range(ROWS_PER_TILE // CH):
        pltpu.async_copy(r0, agg_sh.at[pl.ds(row0 + j * CH, CH)], s0)
    for j in range(ROWS_PER_TILE // CH):
        pltpu.make_async_copy(r0, agg_sh.at[pl.ds(row0 + j * CH, CH)], s0).wait()

    pltpu.async_copy(h_hbm.at[src_u[0]], rows[0], gsem[0])

    plsc.subcore_barrier()

    def wait_gather(p):
        pltpu.make_async_copy(h_hbm.at[src_u[p]], rows[p], gsem[p]).wait()

    def start_scatter(p):
        pltpu.async_copy(rows[p], agg_sh.at[dst_u[p]], ssem[p], add=True)

    def wait_scatter(p):
        pltpu.make_async_copy(rows[p], agg_sh.at[dst_u[p]], ssem[p]).wait()

    def start_gather(c, p):
        unpack(c, p)
        pltpu.async_copy(h_hbm.at[src_u[p]], rows[p], gsem[p])

    units = jnp.where(wid == NW - 1, CPT_LAST // NBUF, CPT // NBUF)

    @pl.loop(0, units - 1)
    def _unit(j):
        u = j * NBUF
        for p in range(NBUF):
            wait_gather(p)
            start_scatter(p)
        for p in range(NBUF):
            wait_scatter(p)
            start_gather(u + NBUF + p, p)

    for p in range(NBUF):
        wait_gather(p)
        start_scatter(p)
    for p in range(NBUF):
        wait_scatter(p)

    plsc.subcore_barrier()

    pltpu.sync_copy(agg_sh.at[pl.ds(row0, ROWS_PER_TILE)],
                    out_hbm.at[cid, pl.ds(row0, ROWS_PER_TILE)])


@jax.jit
def _sc_agg(h, packed):
    k = pl.kernel(
        _agg_body,
        out_type=jax.ShapeDtypeStruct((NC, NPAD, D), jnp.float32),
        mesh=_mesh,
        scratch_types=[
            pltpu.VMEM((EPT_LAST,), jnp.int32),
            pltpu.VMEM((CH, D), jnp.float32),
            pltpu.VMEM((CH, D), jnp.float32),
            pltpu.VMEM((CH, D), jnp.float32),
            pltpu.VMEM((CH, D), jnp.float32),
            pltpu.VMEM((CH,), jnp.int32),
            pltpu.VMEM((CH,), jnp.int32),
            pltpu.VMEM((CH,), jnp.int32),
            pltpu.VMEM((CH,), jnp.int32),
            pltpu.VMEM((CH,), jnp.int32),
            pltpu.VMEM((CH,), jnp.int32),
            pltpu.VMEM((CH,), jnp.int32),
            pltpu.VMEM((CH,), jnp.int32),
            pltpu.VMEM_SHARED((NPAD, D), jnp.float32),
            pltpu.SemaphoreType.DMA,
            pltpu.SemaphoreType.DMA,
            pltpu.SemaphoreType.DMA,
            pltpu.SemaphoreType.DMA,
            pltpu.SemaphoreType.DMA,
            pltpu.SemaphoreType.DMA,
            pltpu.SemaphoreType.DMA,
            pltpu.SemaphoreType.DMA,
        ],
        compiler_params=_sc_params,
    )
    return k(h, packed)


BLK = 2560


def _mm_body(x_ref, w_ref, degp_ref, o_ref):
    deg = jnp.sum(degp_ref[...], axis=0)
    norm = lax.rsqrt(jnp.maximum(deg, 1.0))
    z = jnp.dot(x_ref[...], w_ref[...],
                preferred_element_type=jnp.float32)
    o_ref[...] = z * norm[:, None]


@jax.jit
def _tc_matmul_scale(x, w, degp):
    return pl.pallas_call(
        _mm_body,
        out_shape=jax.ShapeDtypeStruct((NPAD, D), jnp.float32),
        grid=(NPAD // BLK,),
        in_specs=[
            pl.BlockSpec((BLK, D), lambda i: (i, 0)),
            pl.BlockSpec((D, D), lambda i: (0, 0)),
            pl.BlockSpec((NW, BLK), lambda i: (0, i)),
        ],
        out_specs=pl.BlockSpec((BLK, D), lambda i: (i, 0)),
    )(x, w, degp)


def _post_body(p_ref, degp_ref, b_ref, o_ref):
    deg = jnp.sum(degp_ref[...], axis=0)
    norm = lax.rsqrt(jnp.maximum(deg, 1.0))
    agg = p_ref[0] + p_ref[1]
    o_ref[...] = agg * norm[:, None] + b_ref[...][None, :]


@jax.jit
def _tc_post(p, degp, b):
    return pl.pallas_call(
        _post_body,
        out_shape=jax.ShapeDtypeStruct((N, D), jnp.float32),
        grid=(NPAD // BLK,),
        in_specs=[
            pl.BlockSpec((NC, BLK, D), lambda i: (0, i, 0)),
            pl.BlockSpec((NW, BLK), lambda i: (0, i)),
            pl.BlockSpec((D,), lambda i: (0,)),
        ],
        out_specs=pl.BlockSpec((BLK, D), lambda i: (i, 0)),
    )(p, degp, b)


def kernel(ndata, edge_index, W, b):
    ei = edge_index.astype(jnp.int32)
    dego_p, degi_p, packed = _sc_hist(ei)
    h = _tc_matmul_scale(ndata, W, dego_p)
    p = _sc_agg(h, packed)
    return _tc_post(p, degi_p, b)


# speedup vs baseline: 1.0654x; 1.0010x over previous
import dataclasses

import jax
import jax.numpy as jnp
from jax import lax
from jax.experimental import pallas as pl
from jax.experimental.pallas import tpu as pltpu
from jax.experimental.pallas import tpu_sc as plsc

N = 10000
NPAD = 10240
E = 320000
D = 128

NC = 2
NS = 16
NW = NC * NS

EPT = 9984
EPT_LAST = E - EPT * (NW - 1)
CH = 64
CPT = EPT // CH
CPT_LAST = EPT_LAST // CH
NBUF = 4
ROWS_PER_TILE = NPAD // NS

_mesh = plsc.VectorSubcoreMesh(core_axis_name="c", subcore_axis_name="s")

_sc_params = pltpu.CompilerParams()
if "needs_layout_passes" in pltpu.CompilerParams.__dataclass_fields__:
    _sc_params = dataclasses.replace(_sc_params, needs_layout_passes=False)


def _hist_body(ei_hbm, dego_hbm, degi_hbm, pk_hbm,
               idx_sd, pk_v, ho_v, hi_v, sem):
    cid = lax.axis_index("c")
    sid = lax.axis_index("s")
    wid = sid * NC + cid

    zeros16 = jnp.zeros((16,), jnp.float32)
    ones16 = jnp.ones((16,), jnp.float32)
    sixteen = jnp.full((16,), 16, jnp.int32)

    base = wid * EPT
    pltpu.async_copy(ei_hbm.at[:, pl.ds(base, EPT_LAST)], idx_sd, sem)

    @pl.loop(0, NPAD // 64)
    def _zero(i):
        for q in range(4):
            ho_v[pl.ds(i * 64 + q * 16, 16)] = zeros16
            hi_v[pl.ds(i * 64 + q * 16, 16)] = zeros16

    pltpu.make_async_copy(ei_hbm.at[:, pl.ds(base, EPT_LAST)], idx_sd, sem).wait()

    nvec = jnp.where(wid == NW - 1, EPT_LAST // 32, EPT // 32)

    @pl.loop(0, nvec)
    def _accum(i):
        for q in range(2):
            s = idx_sd[0, pl.ds(i * 32 + q * 16, 16)]
            d = idx_sd[1, pl.ds(i * 32 + q * 16, 16)]
            plsc.addupdate_scatter(ho_v, [s], ones16)
            plsc.addupdate_scatter(hi_v, [d], ones16)
            pk_v[pl.ds(i * 32 + q * 16, 16)] = jnp.bitwise_or(
                s, lax.shift_left(d, sixteen))

    pltpu.sync_copy(pk_v, pk_hbm.at[wid])
    pltpu.sync_copy(ho_v, dego_hbm.at[wid])
    pltpu.sync_copy(hi_v, degi_hbm.at[wid])


@jax.jit
def _sc_hist(ei):
    k = pl.kernel(
        _hist_body,
        out_type=(
            jax.ShapeDtypeStruct((NW, NPAD), jnp.float32),
            jax.ShapeDtypeStruct((NW, NPAD), jnp.float32),
            jax.ShapeDtypeStruct((NW, EPT_LAST), jnp.int32),
        ),
        mesh=_mesh,
        scratch_types=[
            pltpu.VMEM((2, EPT_LAST), jnp.int32),
            pltpu.VMEM((EPT_LAST,), jnp.int32),
            pltpu.VMEM((NPAD,), jnp.float32),
            pltpu.VMEM((NPAD,), jnp.float32),
            pltpu.SemaphoreType.DMA,
        ],
        compiler_params=_sc_params,
    )
    return k(ei)


def _agg_body(h_hbm, pk_hbm, out_hbm,
              pkb, r0, r1, r2, r3, su0, su1, su2, su3, du0, du1, du2, du3,
              agg_sh, g0, g1, g2, g3, s0, s1, s2, s3):
    cid = lax.axis_index("c")
    sid = lax.axis_index("s")
    wid = sid * NC + cid

    rows = (r0, r1, r2, r3)
    src_u = (su0, su1, su2, su3)
    dst_u = (du0, du1, du2, du3)
    gsem = (g0, g1, g2, g3)
    ssem = (s0, s1, s2, s3)

    zeros16 = jnp.zeros((16,), jnp.float32)
    mask16 = jnp.full((16,), 0xFFFF, jnp.int32)
    sixteen = jnp.full((16,), 16, jnp.int32)

    def unpack_src(c, p):
        for k8 in range(CH // 16):
            v = pkb[pl.ds(c * CH + k8 * 16, 16)]
            src_u[p][pl.ds(k8 * 16, 16)] = jnp.bitwise_and(v, mask16)

    def unpack_dst(c, p):
        for k8 in range(CH // 16):
            v = pkb[pl.ds(c * CH + k8 * 16, 16)]
            dst_u[p][pl.ds(k8 * 16, 16)] = lax.shift_right_logical(v, sixteen)

    def unpack(c, p):
        unpack_src(c, p)
        unpack_dst(c, p)

    pltpu.sync_copy(pk_hbm.at[wid], pkb)

    for p in range(NBUF):
        unpack(p, p)
    for p in range(1, NBUF):
        pltpu.async_copy(h_hbm.at[src_u[p]], rows[p], gsem[p])

    @pl.loop(0, CH)
    def _zero(r):
        for k8 in range(D // 16):
            r0[r, pl.ds(k8 * 16, 16)] = zeros16

    row0 = sid * ROWS_PER_TILE
    for j in range(ROWS_PER_TILE // CH):
        pltpu.async_copy(r0, agg_sh.at[pl.ds(row0 + j * CH, CH)], s0)
    for j in range(ROWS_PER_TILE // CH):
        pltpu.make_async_copy(r0, agg_sh.at[pl.ds(row0 + j * CH, CH)], s0).wait()

    pltpu.async_copy(h_hbm.at[src_u[0]], rows[0], gsem[0])

    plsc.subcore_barrier()

    def wait_gather(p):
        pltpu.make_async_copy(h_hbm.at[src_u[p]], rows[p], gsem[p]).wait()

    def start_scatter(p):
        pltpu.async_copy(rows[p], agg_sh.at[dst_u[p]], ssem[p], add=True)

    def wait_scatter(p):
        pltpu.make_async_copy(rows[p], agg_sh.at[dst_u[p]], ssem[p]).wait()

    def start_gather(c, p):
        unpack(c, p)
        pltpu.async_copy(h_hbm.at[src_u[p]], rows[p], gsem[p])

    units = jnp.where(wid == NW - 1, CPT_LAST // NBUF, CPT // NBUF)

    @pl.loop(0, units - 1)
    def _unit(j):
        u = j * NBUF
        for p in range(NBUF):
            wait_gather(p)
            start_scatter(p)
            unpack_src(u + NBUF + p, p)
        for p in range(NBUF):
            wait_scatter(p)
            pltpu.async_copy(h_hbm.at[src_u[p]], rows[p], gsem[p])
            unpack_dst(u + NBUF + p, p)

    for p in range(NBUF):
        wait_gather(p)
        start_scatter(p)
    for p in range(NBUF):
        wait_scatter(p)

    plsc.subcore_barrier()

    pltpu.sync_copy(agg_sh.at[pl.ds(row0, ROWS_PER_TILE)],
                    out_hbm.at[cid, pl.ds(row0, ROWS_PER_TILE)])


@jax.jit
def _sc_agg(h, packed):
    k = pl.kernel(
        _agg_body,
        out_type=jax.ShapeDtypeStruct((NC, NPAD, D), jnp.float32),
        mesh=_mesh,
        scratch_types=[
            pltpu.VMEM((EPT_LAST,), jnp.int32),
            pltpu.VMEM((CH, D), jnp.float32),
            pltpu.VMEM((CH, D), jnp.float32),
            pltpu.VMEM((CH, D), jnp.float32),
            pltpu.VMEM((CH, D), jnp.float32),
            pltpu.VMEM((CH,), jnp.int32),
            pltpu.VMEM((CH,), jnp.int32),
            pltpu.VMEM((CH,), jnp.int32),
            pltpu.VMEM((CH,), jnp.int32),
            pltpu.VMEM((CH,), jnp.int32),
            pltpu.VMEM((CH,), jnp.int32),
            pltpu.VMEM((CH,), jnp.int32),
            pltpu.VMEM((CH,), jnp.int32),
            pltpu.VMEM_SHARED((NPAD, D), jnp.float32),
            pltpu.SemaphoreType.DMA,
            pltpu.SemaphoreType.DMA,
            pltpu.SemaphoreType.DMA,
            pltpu.SemaphoreType.DMA,
            pltpu.SemaphoreType.DMA,
            pltpu.SemaphoreType.DMA,
            pltpu.SemaphoreType.DMA,
            pltpu.SemaphoreType.DMA,
        ],
        compiler_params=_sc_params,
    )
    return k(h, packed)


BLK = 2560


def _mm_body(x_ref, w_ref, degp_ref, o_ref):
    deg = jnp.sum(degp_ref[...], axis=0)
    norm = lax.rsqrt(jnp.maximum(deg, 1.0))
    z = jnp.dot(x_ref[...], w_ref[...],
                preferred_element_type=jnp.float32)
    o_ref[...] = z * norm[:, None]


@jax.jit
def _tc_matmul_scale(x, w, degp):
    return pl.pallas_call(
        _mm_body,
        out_shape=jax.ShapeDtypeStruct((NPAD, D), jnp.float32),
        grid=(NPAD // BLK,),
        in_specs=[
            pl.BlockSpec((BLK, D), lambda i: (i, 0)),
            pl.BlockSpec((D, D), lambda i: (0, 0)),
            pl.BlockSpec((NW, BLK), lambda i: (0, i)),
        ],
        out_specs=pl.BlockSpec((BLK, D), lambda i: (i, 0)),
    )(x, w, degp)


def _post_body(p_ref, degp_ref, b_ref, o_ref):
    deg = jnp.sum(degp_ref[...], axis=0)
    norm = lax.rsqrt(jnp.maximum(deg, 1.0))
    agg = p_ref[0] + p_ref[1]
    o_ref[...] = agg * norm[:, None] + b_ref[...][None, :]


@jax.jit
def _tc_post(p, degp, b):
    return pl.pallas_call(
        _post_body,
        out_shape=jax.ShapeDtypeStruct((N, D), jnp.float32),
        grid=(NPAD // BLK,),
        in_specs=[
            pl.BlockSpec((NC, BLK, D), lambda i: (0, i, 0)),
            pl.BlockSpec((NW, BLK), lambda i: (0, i)),
            pl.BlockSpec((D,), lambda i: (0,)),
        ],
        out_specs=pl.BlockSpec((BLK, D), lambda i: (i, 0)),
    )(p, degp, b)


def kernel(ndata, edge_index, W, b):
    ei = edge_index.astype(jnp.int32)
    dego_p, degi_p, packed = _sc_hist(ei)
    h = _tc_matmul_scale(ndata, W, dego_p)
    p = _sc_agg(h, packed)
    return _tc_post(p, degi_p, b)
